# baseline (speedup 1.0000x reference)
import jax
import jax.numpy as jnp
from jax import lax
from jax.experimental import pallas as pl
from jax.experimental.pallas import tpu as pltpu
from jax.experimental.pallas import tpu_sc as plsc

ENT_MAX_LEN = 20
REL_MAX_LEN = 20
PITCH = 24
DIM = 128
BATCH = 4096

NUM_CORES = 2
NUM_SUBCORES = 16
NW = NUM_CORES * NUM_SUBCORES
BW = BATCH // NW
EPG = 2
NG = BW // EPG
NBUF = 3
L = 16


def _clamp_ids(ref_1d, n, hi):
  for c in range(n // L):
    v = ref_1d[pl.ds(c * L, L)]
    ref_1d[pl.ds(c * L, L)] = jnp.minimum(jnp.maximum(v, 0), hi)


def _accum_group(rows, n_tok, acc_v, i0, col0):
  for e in range(EPG):
    r0 = e * PITCH
    for c in range(DIM // L):
      s = rows[r0, pl.ds(c * L, L)]
      for t in range(1, n_tok):
        s = s + rows[r0 + t, pl.ds(c * L, L)]
      acc_v[i0 + e, pl.ds(col0 + c * L, L)] = s


def _fire_tok(tok_h, idx_s, g, stage, sem):
  for e in range(EPG):
    i = g * EPG + e
    pltpu.async_copy(tok_h.at[idx_s[i]], stage[e], sem)


def _drain_tok(tok_h, idx_s, g, stage, tok96, sem, hi):
  for e in range(EPG):
    i = g * EPG + e
    pltpu.make_async_copy(tok_h.at[idx_s[i]], stage[e], sem).wait()
    for off in (0, ENT_MAX_LEN - L):
      v = stage[e][pl.ds(off, L)]
      tok96[pl.ds(e * PITCH + off, L)] = jnp.minimum(jnp.maximum(v, 0), hi)


def _side(tok_h, emb_h, idx_s, n_tok, hi_tok, acc_v, col0,
          stages, toks, rowss, tsems, rsems):
  for b in range(NBUF):
    _fire_tok(tok_h, idx_s, b, stages[b], tsems[b])

  def grp(g, _):
    g0 = g * NBUF
    for b in range(NBUF):
      gg = g0 + b
      _drain_tok(tok_h, idx_s, gg, stages[b], toks[b], tsems[b], hi_tok)
      pltpu.async_copy(emb_h.at[toks[b]], rowss[b], rsems[b])
      nxtf = gg + NBUF

      @pl.when(nxtf < NG)
      def _():
        _fire_tok(tok_h, idx_s, nxtf, stages[b], tsems[b])
    for b in range(NBUF):
      gg = g0 + b
      pltpu.make_async_copy(emb_h.at[toks[b]], rowss[b], rsems[b]).wait()
      _accum_group(rowss[b], n_tok, acc_v, gg * EPG, col0)
    return 0

  lax.fori_loop(0, NG // NBUF, grp, 0)

  for b in range(NG % NBUF):
    gg = (NG // NBUF) * NBUF + b
    _drain_tok(tok_h, idx_s, gg, stages[b], toks[b], tsems[b], hi_tok)
    pltpu.async_copy(emb_h.at[toks[b]], rowss[b], rsems[b]).wait()
    _accum_group(rowss[b], n_tok, acc_v, gg * EPG, col0)


def _body(subj_h, rel_h, etok_h, rtok_h, eemb_h, remb_h, out_h,
          ids_sh, sidx_v, ridx_v, sidx_s, ridx_s, acc_v,
          stages, toks, rowss, tsems, rsems):
  c = lax.axis_index("c")
  s = lax.axis_index("s")
  wid = s * NUM_CORES + c
  base = wid * BW

  pltpu.sync_copy(subj_h.at[pl.ds(base, BW)], sidx_v)
  pltpu.sync_copy(rel_h.at[pl.ds(base, BW)], ridx_v)
  _clamp_ids(sidx_v, BW, 100000 - 1)
  _clamp_ids(ridx_v, BW, 1000 - 1)
  pltpu.sync_copy(sidx_v, ids_sh.at[s, 0])
  pltpu.sync_copy(ridx_v, ids_sh.at[s, 1])
  pltpu.sync_copy(ids_sh.at[s, 0], sidx_s)
  pltpu.sync_copy(ids_sh.at[s, 1], ridx_s)

  zero = jnp.zeros((L,), jnp.int32)
  for b in range(NBUF):
    for k in range(EPG * PITCH // L):
      toks[b][pl.ds(k * L, L)] = zero

  _side(etok_h, eemb_h, sidx_s, ENT_MAX_LEN, 100000 - 1, acc_v, 0,
        stages, toks, rowss, tsems, rsems)
  _side(rtok_h, remb_h, ridx_s, REL_MAX_LEN, 1000 - 1, acc_v, DIM,
        stages, toks, rowss, tsems, rsems)

  pltpu.sync_copy(acc_v, out_h.at[pl.ds(base, BW)])


@jax.jit
def kernel(subj, rel, entity_token_ids, relation_token_ids,
           entity_emb, relation_emb):
  mesh = plsc.VectorSubcoreMesh(core_axis_name="c", subcore_axis_name="s")
  run = pl.kernel(
      _body,
      out_type=jax.ShapeDtypeStruct((BATCH, 2 * DIM), jnp.float32),
      mesh=mesh,
      scratch_types=[
          pltpu.VMEM_SHARED((NUM_SUBCORES, 2, BW), jnp.int32),
          pltpu.VMEM((BW,), jnp.int32),
          pltpu.VMEM((BW,), jnp.int32),
          pltpu.SMEM((BW,), jnp.int32),
          pltpu.SMEM((BW,), jnp.int32),
          pltpu.VMEM((BW, 2 * DIM), jnp.float32),
          [[pltpu.VMEM((ENT_MAX_LEN,), jnp.int32) for _ in range(EPG)]
           for _ in range(NBUF)],
          [pltpu.VMEM((EPG * PITCH,), jnp.int32) for _ in range(NBUF)],
          [pltpu.VMEM((EPG * PITCH, DIM), jnp.float32) for _ in range(NBUF)],
          [pltpu.SemaphoreType.DMA for _ in range(NBUF)],
          [pltpu.SemaphoreType.DMA for _ in range(NBUF)],
      ],
  )
  return run(subj, rel, entity_token_ids, relation_token_ids,
             entity_emb, relation_emb)
